# baseline (speedup 1.0000x reference)
import functools

import jax
import jax.numpy as jnp
from jax.experimental import pallas as pl
from jax.experimental.pallas import tpu as pltpu

_VOCAB = 1000000
_DIM = 32
_CTX = 200
_VB = 10000
_NB = _VOCAB // _VB


def _gather_mean_kernel(idx_ref, e_ref, o_ref):
    i = pl.program_id(0)

    @pl.when(i == 0)
    def _():
        o_ref[...] = jnp.zeros_like(o_ref)

    o_ref[...] += e_ref[0]

    @pl.when(i == _CTX - 1)
    def _():
        o_ref[...] = o_ref[...] * (1.0 / _CTX)


def _proj_kernel(x_ref, w_ref, b_ref, o_ref, y_s, m_s, l_s, z_s):
    p = pl.program_id(0)
    j = pl.program_id(1)

    @pl.when(p == 0)
    def _():
        @pl.when(j == 0)
        def _():
            m_s[...] = jnp.full_like(m_s, -jnp.inf)
            l_s[...] = jnp.zeros_like(l_s)

        y = jax.lax.dot_general(
            x_ref[...], w_ref[...],
            (((1,), (1,)), ((), ())),
            preferred_element_type=jnp.float32,
        ) + b_ref[0]
        y_s[pl.ds(j, 1), :] = y

        m_old = m_s[...]
        m_blk = jnp.max(y, axis=1, keepdims=True)
        m_new = jnp.maximum(m_old, m_blk)
        l_new = l_s[...] * jnp.exp(m_old - m_new) + jnp.sum(
            jnp.exp(y - m_new), axis=1, keepdims=True)
        m_s[...] = m_new
        l_s[...] = l_new

        @pl.when(j == _NB - 1)
        def _():
            z_s[...] = m_new + jnp.log(l_new)

    @pl.when(p == 1)
    def _():
        o_ref[...] = (y_s[pl.ds(j, 1), :] - z_s[...])[None]


@jax.jit
def kernel(inputs, emb_table, lin_w, lin_b):
    emb3 = emb_table.reshape(_VOCAB, 1, _DIM)
    x = pl.pallas_call(
        _gather_mean_kernel,
        grid_spec=pltpu.PrefetchScalarGridSpec(
            num_scalar_prefetch=1,
            grid=(_CTX,),
            in_specs=[
                pl.BlockSpec((1, 1, _DIM), lambda i, idx: (idx[i], 0, 0)),
            ],
            out_specs=pl.BlockSpec((1, _DIM), lambda i, idx: (0, 0)),
        ),
        out_shape=jax.ShapeDtypeStruct((1, _DIM), jnp.float32),
    )(inputs, emb3)

    b3 = lin_b.reshape(_NB, 1, _VB)
    out = pl.pallas_call(
        _proj_kernel,
        grid=(2, _NB),
        in_specs=[
            pl.BlockSpec((1, _DIM), lambda p, j: (0, 0)),
            pl.BlockSpec((_VB, _DIM), lambda p, j: (j * (1 - p), 0)),
            pl.BlockSpec((1, 1, _VB), lambda p, j: (j * (1 - p), 0, 0)),
        ],
        out_specs=pl.BlockSpec((1, 1, _VB), lambda p, j: (j * p, 0, 0)),
        out_shape=jax.ShapeDtypeStruct((_NB, 1, _VB), jnp.float32),
        scratch_shapes=[
            pltpu.VMEM((_NB, _VB), jnp.float32),
            pltpu.VMEM((1, 1), jnp.float32),
            pltpu.VMEM((1, 1), jnp.float32),
            pltpu.VMEM((1, 1), jnp.float32),
        ],
    )(x, lin_w, b3)

    return out.reshape(1, _VOCAB)


# speedup vs baseline: 1.0426x; 1.0426x over previous
import functools

import jax
import jax.numpy as jnp
from jax.experimental import pallas as pl
from jax.experimental.pallas import tpu as pltpu

_VOCAB = 1000000
_DIM = 32
_CTX = 200
_VB = 8192
_NB = -(-_VOCAB // _VB)


def _gather_mean_kernel(idx_ref, e_ref, o_ref):
    i = pl.program_id(0)

    @pl.when(i == 0)
    def _():
        o_ref[...] = jnp.zeros_like(o_ref)

    o_ref[...] += e_ref[0]

    @pl.when(i == _CTX - 1)
    def _():
        o_ref[...] = o_ref[...] * (1.0 / _CTX)


def _proj_kernel(x_ref, w_ref, b_ref, o_ref, y_s, m_s, l_s, z_s):
    p = pl.program_id(0)
    j = pl.program_id(1)

    @pl.when(p == 0)
    def _():
        @pl.when(j == 0)
        def _():
            m_s[...] = jnp.full_like(m_s, -jnp.inf)
            l_s[...] = jnp.zeros_like(l_s)

        y = jax.lax.dot_general(
            x_ref[...], w_ref[...],
            (((1,), (1,)), ((), ())),
            preferred_element_type=jnp.float32,
        ) + b_ref[...][None]
        col = j * _VB + jax.lax.broadcasted_iota(jnp.int32, (1, _VB), 1)
        y = jnp.where(col < _VOCAB, y, -jnp.inf)
        y_s[pl.ds(j, 1), :] = y

        m_old = m_s[...]
        m_blk = jnp.max(y, axis=1, keepdims=True)
        m_new = jnp.maximum(m_old, m_blk)
        l_new = l_s[...] * jnp.exp(m_old - m_new) + jnp.sum(
            jnp.where(col < _VOCAB, jnp.exp(y - m_new), 0.0),
            axis=1, keepdims=True)
        m_s[...] = m_new
        l_s[...] = l_new

        @pl.when(j == _NB - 1)
        def _():
            z_s[...] = m_new + jnp.log(l_new)

    @pl.when(p == 1)
    def _():
        o_ref[...] = y_s[pl.ds(j, 1), :] - z_s[...]


@jax.jit
def kernel(inputs, emb_table, lin_w, lin_b):
    emb3 = emb_table.reshape(_VOCAB, 1, _DIM)
    x = pl.pallas_call(
        _gather_mean_kernel,
        grid_spec=pltpu.PrefetchScalarGridSpec(
            num_scalar_prefetch=1,
            grid=(_CTX,),
            in_specs=[
                pl.BlockSpec((1, 1, _DIM), lambda i, idx: (idx[i], 0, 0)),
            ],
            out_specs=pl.BlockSpec((1, _DIM), lambda i, idx: (0, 0)),
        ),
        out_shape=jax.ShapeDtypeStruct((1, _DIM), jnp.float32),
    )(inputs, emb3)

    out = pl.pallas_call(
        _proj_kernel,
        grid=(2, _NB),
        in_specs=[
            pl.BlockSpec((1, _DIM), lambda p, j: (0, 0)),
            pl.BlockSpec((_VB, _DIM), lambda p, j: (j * (1 - p), 0)),
            pl.BlockSpec((_VB,), lambda p, j: (j * (1 - p),)),
        ],
        out_specs=pl.BlockSpec((1, _VB), lambda p, j: (0, j * p)),
        out_shape=jax.ShapeDtypeStruct((1, _VOCAB), jnp.float32),
        scratch_shapes=[
            pltpu.VMEM((_NB, _VB), jnp.float32),
            pltpu.VMEM((1, 1), jnp.float32),
            pltpu.VMEM((1, 1), jnp.float32),
            pltpu.VMEM((1, 1), jnp.float32),
        ],
    )(x, lin_w, lin_b)

    return out
